# Initial kernel scaffold; baseline (speedup 1.0000x reference)
import functools

import jax
import jax.numpy as jnp
from jax import lax
from jax.experimental import pallas as pl
from jax.experimental.pallas import tpu as pltpu
from jax.experimental.pallas import tpu_sc as plsc

N = 10000
D = 128
E = 320000
EPS = 1e-5

NC = 2
NS = 16
NW = NC * NS
K = 128
CHUNKS = 79
E_PAD = NW * CHUNKS * K
NPAD = N + 16
STRIPE = NPAD // NS

_mesh = plsc.VectorSubcoreMesh(core_axis_name="c", subcore_axis_name="s")


@functools.partial(
    pl.kernel,
    mesh=_mesh,
    out_type=jax.ShapeDtypeStruct((NC, NPAD, 16), jnp.float32),
    scratch_types=[
        pltpu.VMEM((CHUNKS, K), jnp.int32),
        pltpu.VMEM((K, 16), jnp.float32),
        pltpu.VMEM((K, 16), jnp.float32),
        pltpu.VMEM_SHARED((NPAD, 16), jnp.float32),
    ],
)
def _deg_kernel(dst_hbm, ones_hbm, zeros_hbm, out_hbm, dst_v, ones_v, zbuf_v, acc_sh):
    c = lax.axis_index("c")
    s = lax.axis_index("s")
    wid = s * NC + c
    pltpu.sync_copy(dst_hbm.at[wid], dst_v)
    pltpu.sync_copy(ones_hbm, ones_v)
    pltpu.sync_copy(zeros_hbm, zbuf_v)
    base = s * STRIPE
    for i in range(4):
        pltpu.sync_copy(zbuf_v, acc_sh.at[pl.ds(base + i * K, K)])
    pltpu.sync_copy(zbuf_v.at[pl.ds(0, STRIPE - 4 * K)],
                    acc_sh.at[pl.ds(base + 4 * K, STRIPE - 4 * K)])
    plsc.subcore_barrier()

    @pl.loop(0, CHUNKS)
    def _(j):
        pltpu.sync_copy(ones_v, acc_sh.at[dst_v.at[j]], add=True)

    plsc.subcore_barrier()
    pltpu.sync_copy(acc_sh.at[pl.ds(base, STRIPE)],
                    out_hbm.at[c, pl.ds(base, STRIPE)])


@functools.partial(
    pl.kernel,
    mesh=_mesh,
    out_type=jax.ShapeDtypeStruct((NC, NPAD, D), jnp.float32),
    scratch_types=[
        pltpu.VMEM((CHUNKS, K), jnp.int32),
        pltpu.VMEM((CHUNKS, K), jnp.int32),
        pltpu.VMEM((K, D), jnp.float32),
        pltpu.VMEM((K, D), jnp.float32),
        pltpu.VMEM_SHARED((NPAD, D), jnp.float32),
    ],
)
def _scatter_kernel(g_hbm, src_hbm, dst_hbm, zeros_hbm, out_hbm,
                    src_v, dst_v, rows_v, zbuf_v, acc_sh):
    c = lax.axis_index("c")
    s = lax.axis_index("s")
    wid = s * NC + c
    pltpu.sync_copy(src_hbm.at[wid], src_v)
    pltpu.sync_copy(dst_hbm.at[wid], dst_v)
    pltpu.sync_copy(zeros_hbm, zbuf_v)
    base = s * STRIPE
    for i in range(4):
        pltpu.sync_copy(zbuf_v, acc_sh.at[pl.ds(base + i * K, K)])
    pltpu.sync_copy(zbuf_v.at[pl.ds(0, STRIPE - 4 * K)],
                    acc_sh.at[pl.ds(base + 4 * K, STRIPE - 4 * K)])
    plsc.subcore_barrier()

    @pl.loop(0, CHUNKS)
    def _(j):
        pltpu.sync_copy(g_hbm.at[src_v.at[j]], rows_v)
        pltpu.sync_copy(rows_v, acc_sh.at[dst_v.at[j]], add=True)

    plsc.subcore_barrier()
    pltpu.sync_copy(acc_sh.at[pl.ds(base, STRIPE)],
                    out_hbm.at[c, pl.ds(base, STRIPE)])


def _dis_from_deg(deg_ref):
    cnt = deg_ref[0, :, 0:1] + deg_ref[1, :, 0:1]
    return lax.rsqrt(cnt[:N] + 1.0)


def _project(h, w_ref, dis, g_ref):
    g = jnp.dot(h, w_ref[...], preferred_element_type=jnp.float32) * dis
    g_ref[pl.ds(0, N), :] = g
    g_ref[pl.ds(N, NPAD - N), :] = jnp.zeros((NPAD - N, D), jnp.float32)


def _tc0_body(x_ref, w1_ref, b1_ref, w2_ref, b2_ref, cw_ref, deg_ref,
              skip_ref, g_ref):
    dis = _dis_from_deg(deg_ref)
    h = jnp.maximum(jnp.dot(x_ref[...], w1_ref[...],
                            preferred_element_type=jnp.float32) + b1_ref[...], 0.0)
    h = jnp.dot(h, w2_ref[...], preferred_element_type=jnp.float32) + b2_ref[...]
    skip_ref[...] = h
    _project(h, cw_ref, dis, g_ref)


def _tc_mid_body(acc_ref, g_ref, skip_ref, deg_ref, cb_ref, gam_ref, bet_ref,
                 cwn_ref, skip_out_ref, g_out_ref):
    dis = _dis_from_deg(deg_ref)
    agg = acc_ref[0, pl.ds(0, N), :] + acc_ref[1, pl.ds(0, N), :] + g_ref[pl.ds(0, N), :]
    y = dis * agg + cb_ref[...]
    mu = jnp.mean(y, axis=0, keepdims=True)
    var = jnp.mean((y - mu) ** 2, axis=0, keepdims=True)
    y = gam_ref[...] * (y - mu) * lax.rsqrt(var + EPS) + bet_ref[...]
    y = jnp.maximum(y, 0.0) + skip_ref[...]
    skip_out_ref[...] = y
    _project(y, cwn_ref, dis, g_out_ref)


def _tc_final_body(acc_ref, g_ref, deg_ref, cb_ref, out_ref):
    dis = _dis_from_deg(deg_ref)
    agg = acc_ref[0, pl.ds(0, N), :] + acc_ref[1, pl.ds(0, N), :] + g_ref[pl.ds(0, N), :]
    out_ref[...] = dis * agg + cb_ref[...]


_hd = jax.ShapeDtypeStruct((N, D), jnp.float32)
_gd = jax.ShapeDtypeStruct((NPAD, D), jnp.float32)

_tc0 = pl.pallas_call(_tc0_body, out_shape=(_hd, _gd))
_tc_mid = pl.pallas_call(_tc_mid_body, out_shape=(_hd, _gd))
_tc_final = pl.pallas_call(_tc_final_body, out_shape=_hd)


def kernel(x, edge_index, mlp_w1, mlp_b1, mlp_w2, mlp_b2, conv_w0, conv_b0,
           conv_w1, conv_b1, conv_w2, conv_b2, bn_g0, bn_beta0, bn_g1, bn_beta1):
    src = edge_index[0].astype(jnp.int32)
    dst = edge_index[1].astype(jnp.int32)
    pad = jnp.full((E_PAD - E,), N, dtype=jnp.int32)
    src3d = jnp.concatenate([src, pad]).reshape(NW, CHUNKS, K)
    dst3d = jnp.concatenate([dst, pad]).reshape(NW, CHUNKS, K)

    zeros_d = jnp.zeros((K, D), jnp.float32)
    zeros16 = jnp.zeros((K, 16), jnp.float32)
    ones16 = jnp.ones((K, 16), jnp.float32)

    b1 = mlp_b1.reshape(1, D)
    b2 = mlp_b2.reshape(1, D)
    cb0 = conv_b0.reshape(1, D)
    cb1 = conv_b1.reshape(1, D)
    cb2 = conv_b2.reshape(1, D)
    g0_ = bn_g0.reshape(1, D)
    be0 = bn_beta0.reshape(1, D)
    g1_ = bn_g1.reshape(1, D)
    be1 = bn_beta1.reshape(1, D)

    deg = _deg_kernel(dst3d, ones16, zeros16)

    skip, g0 = _tc0(x, mlp_w1, b1, mlp_w2, b2, conv_w0, deg)
    acc0 = _scatter_kernel(g0, src3d, dst3d, zeros_d)
    skip1, g1 = _tc_mid(acc0, g0, skip, deg, cb0, g0_, be0, conv_w1)
    acc1 = _scatter_kernel(g1, src3d, dst3d, zeros_d)
    skip2, g2 = _tc_mid(acc1, g1, skip1, deg, cb1, g1_, be1, conv_w2)
    acc2 = _scatter_kernel(g2, src3d, dst3d, zeros_d)
    return _tc_final(acc2, g2, deg, cb2)


# speedup vs baseline: 10.7724x; 10.7724x over previous
import functools

import jax
import jax.numpy as jnp
from jax import lax
from jax.experimental import pallas as pl
from jax.experimental.pallas import tpu as pltpu
from jax.experimental.pallas import tpu_sc as plsc

N = 10000
D = 128
E = 320000
EPS = 1e-5

NC = 2
NS = 16
NW = NC * NS
K = 128
CHUNKS = 79
E_PAD = NW * CHUNKS * K
NPAD = 10240
STRIPE = NPAD // NS

_mesh = plsc.VectorSubcoreMesh(core_axis_name="c", subcore_axis_name="s")


@functools.partial(
    pl.kernel,
    mesh=_mesh,
    out_type=jax.ShapeDtypeStruct((NC, NPAD, 16), jnp.float32),
    scratch_types=[
        pltpu.VMEM((CHUNKS, K), jnp.int32),
        pltpu.VMEM((K, 16), jnp.float32),
        pltpu.VMEM_SHARED((NPAD, 16), jnp.float32),
    ],
)
def _deg_kernel(dst_hbm, ones_hbm, zeros_hbm, out_hbm, dst_v, ones_v, acc_sh):
    c = lax.axis_index("c")
    s = lax.axis_index("s")
    wid = s * NC + c
    pltpu.sync_copy(dst_hbm.at[wid], dst_v)
    pltpu.sync_copy(ones_hbm, ones_v)
    base = s * STRIPE
    for i in range(STRIPE // K):
        pltpu.sync_copy(zeros_hbm, acc_sh.at[pl.ds(base + i * K, K)])
    plsc.subcore_barrier()

    @pl.loop(0, CHUNKS)
    def _(j):
        pltpu.sync_copy(ones_v, acc_sh.at[dst_v.at[j]], add=True)

    plsc.subcore_barrier()
    pltpu.sync_copy(acc_sh.at[pl.ds(base, STRIPE)],
                    out_hbm.at[c, pl.ds(base, STRIPE)])


@functools.partial(
    pl.kernel,
    mesh=_mesh,
    out_type=jax.ShapeDtypeStruct((NC, NPAD, D), jnp.float32),
    scratch_types=[
        pltpu.VMEM((CHUNKS, K), jnp.int32),
        pltpu.VMEM((CHUNKS, K), jnp.int32),
        pltpu.VMEM((K, D), jnp.float32),
        pltpu.VMEM_SHARED((NPAD, D), jnp.float32),
    ],
)
def _scatter_kernel(g_hbm, src_hbm, dst_hbm, zeros_hbm, out_hbm,
                    src_v, dst_v, rows_v, acc_sh):
    c = lax.axis_index("c")
    s = lax.axis_index("s")
    wid = s * NC + c
    pltpu.sync_copy(src_hbm.at[wid], src_v)
    pltpu.sync_copy(dst_hbm.at[wid], dst_v)
    base = s * STRIPE
    for i in range(STRIPE // K):
        pltpu.sync_copy(zeros_hbm, acc_sh.at[pl.ds(base + i * K, K)])
    plsc.subcore_barrier()

    @pl.loop(0, CHUNKS)
    def _(j):
        pltpu.sync_copy(g_hbm.at[src_v.at[j]], rows_v)
        pltpu.sync_copy(rows_v, acc_sh.at[dst_v.at[j]], add=True)

    plsc.subcore_barrier()
    pltpu.sync_copy(acc_sh.at[pl.ds(base, STRIPE)],
                    out_hbm.at[c, pl.ds(base, STRIPE)])


def _dis_from_deg(deg_ref):
    cnt = deg_ref[0, :, 0:1] + deg_ref[1, :, 0:1]
    return lax.rsqrt(cnt[:N] + 1.0)


def _project(h, w_ref, dis, g_ref):
    g = jnp.dot(h, w_ref[...], preferred_element_type=jnp.float32) * dis
    g_ref[pl.ds(0, N), :] = g
    g_ref[pl.ds(N, NPAD - N), :] = jnp.zeros((NPAD - N, D), jnp.float32)


def _tc0_body(x_ref, w1_ref, b1_ref, w2_ref, b2_ref, cw_ref, deg_ref,
              skip_ref, g_ref):
    dis = _dis_from_deg(deg_ref)
    h = jnp.maximum(jnp.dot(x_ref[...], w1_ref[...],
                            preferred_element_type=jnp.float32) + b1_ref[...], 0.0)
    h = jnp.dot(h, w2_ref[...], preferred_element_type=jnp.float32) + b2_ref[...]
    skip_ref[...] = h
    _project(h, cw_ref, dis, g_ref)


def _tc_mid_body(acc_ref, g_ref, skip_ref, deg_ref, cb_ref, gam_ref, bet_ref,
                 cwn_ref, skip_out_ref, g_out_ref):
    dis = _dis_from_deg(deg_ref)
    agg = acc_ref[0, pl.ds(0, N), :] + acc_ref[1, pl.ds(0, N), :] + g_ref[pl.ds(0, N), :]
    y = dis * agg + cb_ref[...]
    mu = jnp.mean(y, axis=0, keepdims=True)
    var = jnp.mean((y - mu) ** 2, axis=0, keepdims=True)
    y = gam_ref[...] * (y - mu) * lax.rsqrt(var + EPS) + bet_ref[...]
    y = jnp.maximum(y, 0.0) + skip_ref[...]
    skip_out_ref[...] = y
    _project(y, cwn_ref, dis, g_out_ref)


def _tc_final_body(acc_ref, g_ref, deg_ref, cb_ref, out_ref):
    dis = _dis_from_deg(deg_ref)
    agg = acc_ref[0, pl.ds(0, N), :] + acc_ref[1, pl.ds(0, N), :] + g_ref[pl.ds(0, N), :]
    out_ref[...] = dis * agg + cb_ref[...]


_hd = jax.ShapeDtypeStruct((N, D), jnp.float32)
_gd = jax.ShapeDtypeStruct((NPAD, D), jnp.float32)

_tc0 = pl.pallas_call(_tc0_body, out_shape=(_hd, _gd))
_tc_mid = pl.pallas_call(_tc_mid_body, out_shape=(_hd, _gd))
_tc_final = pl.pallas_call(_tc_final_body, out_shape=_hd)


def kernel(x, edge_index, mlp_w1, mlp_b1, mlp_w2, mlp_b2, conv_w0, conv_b0,
           conv_w1, conv_b1, conv_w2, conv_b2, bn_g0, bn_beta0, bn_g1, bn_beta1):
    src = edge_index[0].astype(jnp.int32)
    dst = edge_index[1].astype(jnp.int32)
    pad = jnp.full((E_PAD - E,), N, dtype=jnp.int32)
    src3d = jnp.concatenate([src, pad]).reshape(NW, CHUNKS, K)
    dst3d = jnp.concatenate([dst, pad]).reshape(NW, CHUNKS, K)

    zeros_d = jnp.zeros((K, D), jnp.float32)
    zeros16 = jnp.zeros((K, 16), jnp.float32)
    ones16 = jnp.ones((K, 16), jnp.float32)

    b1 = mlp_b1.reshape(1, D)
    b2 = mlp_b2.reshape(1, D)
    cb0 = conv_b0.reshape(1, D)
    cb1 = conv_b1.reshape(1, D)
    cb2 = conv_b2.reshape(1, D)
    g0_ = bn_g0.reshape(1, D)
    be0 = bn_beta0.reshape(1, D)
    g1_ = bn_g1.reshape(1, D)
    be1 = bn_beta1.reshape(1, D)

    deg = _deg_kernel(dst3d, ones16, zeros16)

    skip, g0 = _tc0(x, mlp_w1, b1, mlp_w2, b2, conv_w0, deg)
    acc0 = _scatter_kernel(g0, src3d, dst3d, zeros_d)
    skip1, g1 = _tc_mid(acc0, g0, skip, deg, cb0, g0_, be0, conv_w1)
    acc1 = _scatter_kernel(g1, src3d, dst3d, zeros_d)
    skip2, g2 = _tc_mid(acc1, g1, skip1, deg, cb1, g1_, be1, conv_w2)
    acc2 = _scatter_kernel(g2, src3d, dst3d, zeros_d)
    return _tc_final(acc2, g2, deg, cb2)


# speedup vs baseline: 11.8562x; 1.1006x over previous
import functools

import jax
import jax.numpy as jnp
from jax import lax
from jax.experimental import pallas as pl
from jax.experimental.pallas import tpu as pltpu
from jax.experimental.pallas import tpu_sc as plsc

N = 10000
D = 128
E = 320000
EPS = 1e-5

NC = 2
NS = 16
NW = NC * NS
K = 128
CHUNKS = 79
E_PAD = NW * CHUNKS * K
NPAD = 10240
STRIPE = NPAD // NS

_mesh = plsc.VectorSubcoreMesh(core_axis_name="c", subcore_axis_name="s")


@functools.partial(
    pl.kernel,
    mesh=_mesh,
    out_type=jax.ShapeDtypeStruct((NC, NPAD, 16), jnp.float32),
    scratch_types=[
        pltpu.VMEM((CHUNKS, K), jnp.int32),
        pltpu.VMEM((K, 16), jnp.float32),
        pltpu.VMEM_SHARED((NPAD, 16), jnp.float32),
    ],
)
def _deg_kernel(dst_hbm, ones_hbm, zeros_hbm, out_hbm, dst_v, ones_v, acc_sh):
    c = lax.axis_index("c")
    s = lax.axis_index("s")
    wid = s * NC + c
    pltpu.sync_copy(dst_hbm.at[wid], dst_v)
    pltpu.sync_copy(ones_hbm, ones_v)
    base = s * STRIPE
    for i in range(STRIPE // K):
        pltpu.sync_copy(zeros_hbm, acc_sh.at[pl.ds(base + i * K, K)])
    plsc.subcore_barrier()

    @pl.loop(0, CHUNKS)
    def _(j):
        pltpu.sync_copy(ones_v, acc_sh.at[dst_v.at[j]], add=True)

    plsc.subcore_barrier()
    pltpu.sync_copy(acc_sh.at[pl.ds(base, STRIPE)],
                    out_hbm.at[c, pl.ds(base, STRIPE)])


@functools.partial(
    pl.kernel,
    mesh=_mesh,
    out_type=jax.ShapeDtypeStruct((NC, NPAD, D), jnp.float32),
    scratch_types=[
        pltpu.VMEM((1, K), jnp.int32),
        pltpu.VMEM((1, K), jnp.int32),
        pltpu.VMEM((CHUNKS, K), jnp.int32),
        pltpu.VMEM((K, D), jnp.float32),
        pltpu.VMEM((K, D), jnp.float32),
        pltpu.VMEM_SHARED((NPAD, D), jnp.float32),
        pltpu.SemaphoreType.DMA,
        pltpu.SemaphoreType.DMA,
    ],
)
def _scatter_kernel(g_hbm, src_hbm, dst_hbm, zeros_hbm, out_hbm,
                    sb0, sb1, dst_v, rows0, rows1, acc_sh, semg0, semg1):
    c = lax.axis_index("c")
    s = lax.axis_index("s")
    wid = s * NC + c
    cbase = wid * CHUNKS
    pltpu.sync_copy(dst_hbm.at[wid], dst_v)
    base = s * STRIPE
    for i in range(STRIPE // K):
        pltpu.sync_copy(zeros_hbm, acc_sh.at[pl.ds(base + i * K, K)])
    plsc.subcore_barrier()

    pltpu.sync_copy(src_hbm.at[cbase], sb0)
    pltpu.async_copy(g_hbm.at[sb0.at[0]], rows0, semg0)

    @pl.loop(0, CHUNKS - 1, step=2)
    def _(j):
        pltpu.sync_copy(src_hbm.at[cbase + j + 1], sb1)
        pltpu.async_copy(g_hbm.at[sb1.at[0]], rows1, semg1)
        pltpu.make_async_copy(g_hbm.at[sb0.at[0]], rows0, semg0).wait()
        pltpu.sync_copy(rows0, acc_sh.at[dst_v.at[j]], add=True)
        pltpu.sync_copy(src_hbm.at[cbase + j + 2], sb0)
        pltpu.async_copy(g_hbm.at[sb0.at[0]], rows0, semg0)
        pltpu.make_async_copy(g_hbm.at[sb1.at[0]], rows1, semg1).wait()
        pltpu.sync_copy(rows1, acc_sh.at[dst_v.at[j + 1]], add=True)

    pltpu.make_async_copy(g_hbm.at[sb0.at[0]], rows0, semg0).wait()
    pltpu.sync_copy(rows0, acc_sh.at[dst_v.at[CHUNKS - 1]], add=True)

    plsc.subcore_barrier()
    pltpu.sync_copy(acc_sh.at[pl.ds(base, STRIPE)],
                    out_hbm.at[c, pl.ds(base, STRIPE)])


def _dis_from_deg(deg_ref):
    cnt = deg_ref[0, :, 0:1] + deg_ref[1, :, 0:1]
    return lax.rsqrt(cnt[:N] + 1.0)


def _project(h, w_ref, dis, g_ref):
    g = jnp.dot(h, w_ref[...], preferred_element_type=jnp.float32) * dis
    g_ref[pl.ds(0, N), :] = g
    g_ref[pl.ds(N, NPAD - N), :] = jnp.zeros((NPAD - N, D), jnp.float32)


def _tc0_body(x_ref, w1_ref, b1_ref, w2_ref, b2_ref, cw_ref, deg_ref,
              skip_ref, g_ref):
    dis = _dis_from_deg(deg_ref)
    h = jnp.maximum(jnp.dot(x_ref[...], w1_ref[...],
                            preferred_element_type=jnp.float32) + b1_ref[...], 0.0)
    h = jnp.dot(h, w2_ref[...], preferred_element_type=jnp.float32) + b2_ref[...]
    skip_ref[...] = h
    _project(h, cw_ref, dis, g_ref)


def _tc_mid_body(acc_ref, g_ref, skip_ref, deg_ref, cb_ref, gam_ref, bet_ref,
                 cwn_ref, skip_out_ref, g_out_ref):
    dis = _dis_from_deg(deg_ref)
    agg = acc_ref[0, pl.ds(0, N), :] + acc_ref[1, pl.ds(0, N), :] + g_ref[pl.ds(0, N), :]
    y = dis * agg + cb_ref[...]
    mu = jnp.mean(y, axis=0, keepdims=True)
    var = jnp.mean((y - mu) ** 2, axis=0, keepdims=True)
    y = gam_ref[...] * (y - mu) * lax.rsqrt(var + EPS) + bet_ref[...]
    y = jnp.maximum(y, 0.0) + skip_ref[...]
    skip_out_ref[...] = y
    _project(y, cwn_ref, dis, g_out_ref)


def _tc_final_body(acc_ref, g_ref, deg_ref, cb_ref, out_ref):
    dis = _dis_from_deg(deg_ref)
    agg = acc_ref[0, pl.ds(0, N), :] + acc_ref[1, pl.ds(0, N), :] + g_ref[pl.ds(0, N), :]
    out_ref[...] = dis * agg + cb_ref[...]


_hd = jax.ShapeDtypeStruct((N, D), jnp.float32)
_gd = jax.ShapeDtypeStruct((NPAD, D), jnp.float32)

_tc0 = pl.pallas_call(_tc0_body, out_shape=(_hd, _gd))
_tc_mid = pl.pallas_call(_tc_mid_body, out_shape=(_hd, _gd))
_tc_final = pl.pallas_call(_tc_final_body, out_shape=_hd)


def kernel(x, edge_index, mlp_w1, mlp_b1, mlp_w2, mlp_b2, conv_w0, conv_b0,
           conv_w1, conv_b1, conv_w2, conv_b2, bn_g0, bn_beta0, bn_g1, bn_beta1):
    src = edge_index[0].astype(jnp.int32)
    dst = edge_index[1].astype(jnp.int32)
    pad = jnp.full((E_PAD - E,), N, dtype=jnp.int32)
    src3d = jnp.concatenate([src, pad]).reshape(NW * CHUNKS, 1, K)
    dst3d = jnp.concatenate([dst, pad]).reshape(NW, CHUNKS, K)

    zeros_d = jnp.zeros((K, D), jnp.float32)
    zeros16 = jnp.zeros((K, 16), jnp.float32)
    ones16 = jnp.ones((K, 16), jnp.float32)

    b1 = mlp_b1.reshape(1, D)
    b2 = mlp_b2.reshape(1, D)
    cb0 = conv_b0.reshape(1, D)
    cb1 = conv_b1.reshape(1, D)
    cb2 = conv_b2.reshape(1, D)
    g0_ = bn_g0.reshape(1, D)
    be0 = bn_beta0.reshape(1, D)
    g1_ = bn_g1.reshape(1, D)
    be1 = bn_beta1.reshape(1, D)

    deg = _deg_kernel(dst3d, ones16, zeros16)

    skip, g0 = _tc0(x, mlp_w1, b1, mlp_w2, b2, conv_w0, deg)
    acc0 = _scatter_kernel(g0, src3d, dst3d, zeros_d)
    skip1, g1 = _tc_mid(acc0, g0, skip, deg, cb0, g0_, be0, conv_w1)
    acc1 = _scatter_kernel(g1, src3d, dst3d, zeros_d)
    skip2, g2 = _tc_mid(acc1, g1, skip1, deg, cb1, g1_, be1, conv_w2)
    acc2 = _scatter_kernel(g2, src3d, dst3d, zeros_d)
    return _tc_final(acc2, g2, deg, cb2)
